# Initial kernel scaffold; baseline (speedup 1.0000x reference)
import functools

import jax
import jax.numpy as jnp
from jax import lax
from jax.experimental import pallas as pl
from jax.experimental.pallas import tpu as pltpu
from jax.experimental.pallas import tpu_sc as plsc

VOCAB = 1000000
EMBED_DIM = 64
BATCH = 16384
HIST = 50

_B = BATCH * HIST
_NC, _NS = 2, 16
_NW = _NC * _NS
_ROWS_PER_W = _B // _NW
_IDXW = 128
_K = 8
_CHUNK = _K * _IDXW
_N_CHUNKS = _ROWS_PER_W // _CHUNK


def _gather_kernel(idx_hbm, table_hbm, out_hbm, idx_v, rows_v, sem):
    wid = lax.axis_index("s") * _NC + lax.axis_index("c")
    row0 = wid * _ROWS_PER_W
    irow0 = row0 // _IDXW

    def body(chunk, carry):
        pltpu.sync_copy(idx_hbm.at[pl.ds(irow0 + chunk * _K, _K)], idx_v)
        copies = []
        for j in range(_K):
            copies.append(
                pltpu.async_copy(
                    table_hbm.at[idx_v.at[j]],
                    rows_v.at[pl.ds(j * _IDXW, _IDXW)],
                    sem,
                )
            )
        for c in copies:
            c.wait()
        pltpu.sync_copy(
            rows_v, out_hbm.at[pl.ds(row0 + chunk * _CHUNK, _CHUNK)]
        )
        return carry

    lax.fori_loop(0, _N_CHUNKS, body, 0)


def _embed_lookup(idx2d, table):
    mesh = plsc.VectorSubcoreMesh(core_axis_name="c", subcore_axis_name="s")
    k = functools.partial(
        pl.kernel,
        mesh=mesh,
        out_type=jax.ShapeDtypeStruct((_B, EMBED_DIM), jnp.float32),
        scratch_types=[
            pltpu.VMEM((_K, _IDXW), jnp.int32),
            pltpu.VMEM((_CHUNK, EMBED_DIM), jnp.float32),
            pltpu.SemaphoreType.DMA,
        ],
    )(_gather_kernel)
    return k(idx2d, table)


def kernel(x, table):
    idx2d = x.reshape(_B // _IDXW, _IDXW).astype(jnp.int32)
    out = _embed_lookup(idx2d, table)
    return out.reshape(BATCH, HIST, EMBED_DIM)


# speedup vs baseline: 1.8439x; 1.8439x over previous
import functools

import jax
import jax.numpy as jnp
from jax import lax
from jax.experimental import pallas as pl
from jax.experimental.pallas import tpu as pltpu
from jax.experimental.pallas import tpu_sc as plsc

VOCAB = 1000000
EMBED_DIM = 64
BATCH = 16384
HIST = 50

_B = BATCH * HIST
_NC, _NS = 2, 16
_NW = _NC * _NS
_ROWS_PER_W = _B // _NW
_IDXW = 128
_K = 8
_CHUNK = _K * _IDXW
_N_CHUNKS = _ROWS_PER_W // _CHUNK


def _gather_kernel(idx_hbm, table_hbm, out_hbm, idx_v, rows_v, sem):
    wid = lax.axis_index("s") * _NC + lax.axis_index("c")
    row0 = wid * _ROWS_PER_W
    irow0 = row0 // _IDXW

    def body(chunk, carry):
        ioff = pl.multiple_of(irow0 + chunk * _K, 8)
        pltpu.sync_copy(idx_hbm.at[pl.ds(ioff, _K)], idx_v)
        copies = []
        for j in range(_K):
            copies.append(
                pltpu.async_copy(
                    table_hbm.at[idx_v.at[j]],
                    rows_v.at[pl.ds(j * _IDXW, _IDXW)],
                    sem,
                )
            )
        for c in copies:
            c.wait()
        ooff = pl.multiple_of(row0 + chunk * _CHUNK, 8)
        pltpu.sync_copy(rows_v, out_hbm.at[pl.ds(ooff, _CHUNK)])
        return carry

    lax.fori_loop(0, _N_CHUNKS, body, 0)


def _embed_lookup(idx2d, table):
    mesh = plsc.VectorSubcoreMesh(core_axis_name="c", subcore_axis_name="s")
    k = functools.partial(
        pl.kernel,
        mesh=mesh,
        out_type=jax.ShapeDtypeStruct((_B, EMBED_DIM), jnp.float32),
        scratch_types=[
            pltpu.VMEM((_K, _IDXW), jnp.int32),
            pltpu.VMEM((_CHUNK, EMBED_DIM), jnp.float32),
            pltpu.SemaphoreType.DMA,
        ],
        compiler_params=pltpu.CompilerParams(use_tc_tiling_on_sc=False),
    )(_gather_kernel)
    return k(idx2d, table)


def kernel(x, table):
    idx2d = x.reshape(_B // _IDXW, _IDXW).astype(jnp.int32)
    out = _embed_lookup(idx2d, table)
    return out.reshape(BATCH, HIST, EMBED_DIM)


# speedup vs baseline: 1.8737x; 1.0161x over previous
import functools

import jax
import jax.numpy as jnp
from jax import lax
from jax.experimental import pallas as pl
from jax.experimental.pallas import tpu as pltpu
from jax.experimental.pallas import tpu_sc as plsc

VOCAB = 1000000
EMBED_DIM = 64
BATCH = 16384
HIST = 50

_B = BATCH * HIST
_NC, _NS = 2, 16
_NW = _NC * _NS
_ROWS_PER_W = _B // _NW
_IDXW = 128
_K = 5
_CHUNK = _K * _IDXW
_N_CHUNKS = _ROWS_PER_W // _CHUNK
_N_PAIRS = _N_CHUNKS // 2


def _gather_kernel(idx_hbm, table_hbm, out_hbm, idx_v, rows0, rows1, gsem, osem):
    wid = lax.axis_index("s") * _NC + lax.axis_index("c")
    row0 = wid * _ROWS_PER_W
    irow0 = row0 // _IDXW

    pltpu.sync_copy(
        idx_hbm.at[pl.ds(pl.multiple_of(irow0, 8), _ROWS_PER_W // _IDXW)], idx_v
    )

    def out_slice(chunk):
        off = pl.multiple_of(row0 + chunk * _CHUNK, 8)
        return out_hbm.at[pl.ds(off, _CHUNK)]

    def fire_gathers(chunk, rows_v):
        return [
            pltpu.async_copy(
                table_hbm.at[idx_v.at[chunk * _K + j]],
                rows_v.at[pl.ds(j * _IDXW, _IDXW)],
                gsem,
            )
            for j in range(_K)
        ]

    def wait_out(chunk, rows_v):
        pltpu.make_async_copy(rows_v, out_slice(chunk), osem).wait()

    def pair(i2, first):
        c0 = 2 * i2
        c1 = c0 + 1
        g0 = fire_gathers(c0, rows0)
        for c in g0:
            c.wait()
        if not first:
            wait_out(c0 - 1, rows1)
        pltpu.async_copy(rows0, out_slice(c0), osem)

        g1 = fire_gathers(c1, rows1)
        for c in g1:
            c.wait()
        wait_out(c0, rows0)
        pltpu.async_copy(rows1, out_slice(c1), osem)

    pair(0, first=True)

    def body(i2, carry):
        pair(i2, first=False)
        return carry

    lax.fori_loop(1, _N_PAIRS, body, 0)

    wait_out(_N_CHUNKS - 1, rows1)


def _embed_lookup(idx2d, table):
    mesh = plsc.VectorSubcoreMesh(core_axis_name="c", subcore_axis_name="s")
    k = functools.partial(
        pl.kernel,
        mesh=mesh,
        out_type=jax.ShapeDtypeStruct((_B, EMBED_DIM), jnp.float32),
        scratch_types=[
            pltpu.VMEM((_ROWS_PER_W // _IDXW, _IDXW), jnp.int32),
            pltpu.VMEM((_CHUNK, EMBED_DIM), jnp.float32),
            pltpu.VMEM((_CHUNK, EMBED_DIM), jnp.float32),
            pltpu.SemaphoreType.DMA,
            pltpu.SemaphoreType.DMA,
        ],
        compiler_params=pltpu.CompilerParams(use_tc_tiling_on_sc=False),
    )(_gather_kernel)
    return k(idx2d, table)


def kernel(x, table):
    idx2d = x.reshape(_B // _IDXW, _IDXW).astype(jnp.int32)
    out = _embed_lookup(idx2d, table)
    return out.reshape(BATCH, HIST, EMBED_DIM)


# speedup vs baseline: 1.8768x; 1.0016x over previous
import functools

import jax
import jax.numpy as jnp
from jax import lax
from jax.experimental import pallas as pl
from jax.experimental.pallas import tpu as pltpu
from jax.experimental.pallas import tpu_sc as plsc

VOCAB = 1000000
EMBED_DIM = 64
BATCH = 16384
HIST = 50

_B = BATCH * HIST
_NC, _NS = 2, 16
_NW = _NC * _NS
_ROWS_PER_W = _B // _NW
_IDXW = 128
_K = 5
_CHUNK = _K * _IDXW
_N_CHUNKS = _ROWS_PER_W // _CHUNK
_N_PAIRS = _N_CHUNKS // 2


def _gather_kernel(idx_hbm, table_hbm, out_hbm, idx_v, rows0, rows1, gsem):
    wid = lax.axis_index("s") * _NC + lax.axis_index("c")
    row0 = wid * _ROWS_PER_W
    irow0 = row0 // _IDXW

    pltpu.sync_copy(
        idx_hbm.at[pl.ds(pl.multiple_of(irow0, 8), _ROWS_PER_W // _IDXW)], idx_v
    )

    def out_slice(chunk):
        off = pl.multiple_of(row0 + chunk * _CHUNK, 8)
        return out_hbm.at[pl.ds(off, _CHUNK)]

    def fire_gathers(chunk, rows_v):
        for j in range(_K):
            pltpu.async_copy(
                table_hbm.at[idx_v.at[chunk * _K + j]],
                rows_v.at[pl.ds(j * _IDXW, _IDXW)],
                gsem,
            )

    def drain_chunk(rows_v):
        pltpu.make_async_copy(
            table_hbm.at[pl.ds(0, _CHUNK)], rows_v, gsem
        ).wait()

    fire_gathers(0, rows0)

    def pair(i2, fire_ahead):
        c0 = 2 * i2
        c1 = c0 + 1
        fire_gathers(c1, rows1)
        drain_chunk(rows0)
        pltpu.sync_copy(rows0, out_slice(c0))
        if fire_ahead:
            fire_gathers(c0 + 2, rows0)
        drain_chunk(rows1)
        pltpu.sync_copy(rows1, out_slice(c1))

    def body(i2, carry):
        pair(i2, fire_ahead=True)
        return carry

    lax.fori_loop(0, _N_PAIRS - 1, body, 0)
    pair(_N_PAIRS - 1, fire_ahead=False)


def _embed_lookup(idx2d, table):
    mesh = plsc.VectorSubcoreMesh(core_axis_name="c", subcore_axis_name="s")
    k = functools.partial(
        pl.kernel,
        mesh=mesh,
        out_type=jax.ShapeDtypeStruct((_B, EMBED_DIM), jnp.float32),
        scratch_types=[
            pltpu.VMEM((_ROWS_PER_W // _IDXW, _IDXW), jnp.int32),
            pltpu.VMEM((_CHUNK, EMBED_DIM), jnp.float32),
            pltpu.VMEM((_CHUNK, EMBED_DIM), jnp.float32),
            pltpu.SemaphoreType.DMA,
        ],
        compiler_params=pltpu.CompilerParams(use_tc_tiling_on_sc=False),
    )(_gather_kernel)
    return k(idx2d, table)


def kernel(x, table):
    idx2d = x.reshape(_B // _IDXW, _IDXW).astype(jnp.int32)
    out = _embed_lookup(idx2d, table)
    return out.reshape(BATCH, HIST, EMBED_DIM)
